# Initial kernel scaffold; baseline (speedup 1.0000x reference)
import jax
import jax.numpy as jnp
from jax.experimental import pallas as pl


def kernel(batch_input, h0, c0, table, Wih0, Whh0, bih0, bhh0, Wih1, Whh1, bih1, bhh1):
    raise NotImplementedError("write your pallas kernel here")


# speedup vs baseline: 14.0708x; 14.0708x over previous
import functools

import jax
import jax.numpy as jnp
from jax import lax
from jax.experimental import pallas as pl
from jax.experimental.pallas import tpu as pltpu
from jax.experimental.pallas import tpu_sc as plsc

_V = 1000000
_H = 128
_B = 1024
_T = 200
_L = 2

_NW = 32
_CH = 128
_NCH = (_B * _T) // (_NW * _CH)


def _sc_gather(table, idx3):
    mesh = plsc.VectorSubcoreMesh(core_axis_name="c", subcore_axis_name="s")

    @functools.partial(
        pl.kernel,
        mesh=mesh,
        out_type=jax.ShapeDtypeStruct((_B * _T, _H), jnp.float32),
        scratch_types=[
            pltpu.VMEM((_NCH, _CH), jnp.int32),
            pltpu.VMEM((_CH, _H), jnp.float32),
            pltpu.SemaphoreType.DMA,
        ],
    )
    def gather_k(table_hbm, idx_hbm, out_hbm, idx_v, buf_v, sem):
        wid = lax.axis_index("s") * 2 + lax.axis_index("c")
        pltpu.sync_copy(idx_hbm.at[wid], idx_v)
        base = wid * (_NCH * _CH)

        def body(j, carry):
            pltpu.async_copy(table_hbm.at[idx_v.at[j]], buf_v, sem).wait()
            pltpu.sync_copy(buf_v, out_hbm.at[pl.ds(base + j * _CH, _CH)])
            return carry

        lax.fori_loop(0, _NCH, body, 0)

    return gather_k(table, idx3)


def _lstm_body(x_ref, h0_ref, c0_ref, wi0_ref, wh0_ref, b0_ref,
               wi1_ref, wh1_ref, b1_ref,
               out_ref, hn_ref, cn_ref,
               h0s, c0s, h1s, c1s):
    t = pl.program_id(0)

    @pl.when(t == 0)
    def _init():
        h0s[...] = h0_ref[0]
        c0s[...] = c0_ref[0]
        h1s[...] = h0_ref[1]
        c1s[...] = c0_ref[1]

    def cell(xt, hs, cs, wi, wh, b):
        g = jnp.dot(xt, wi, preferred_element_type=jnp.float32)
        g = g + jnp.dot(hs, wh, preferred_element_type=jnp.float32)
        g = g + b
        i = jax.nn.sigmoid(g[:, 0 * _H:1 * _H])
        f = jax.nn.sigmoid(g[:, 1 * _H:2 * _H])
        gg = jnp.tanh(g[:, 2 * _H:3 * _H])
        o = jax.nn.sigmoid(g[:, 3 * _H:4 * _H])
        c = f * cs + i * gg
        h = o * jnp.tanh(c)
        return h, c

    xt = x_ref[0]
    h0n, c0n = cell(xt, h0s[...], c0s[...], wi0_ref[...], wh0_ref[...],
                    b0_ref[...])
    h0s[...] = h0n
    c0s[...] = c0n
    h1n, c1n = cell(h0n, h1s[...], c1s[...], wi1_ref[...], wh1_ref[...],
                    b1_ref[...])
    h1s[...] = h1n
    c1s[...] = c1n
    out_ref[0] = h1n

    @pl.when(t == _T - 1)
    def _fin():
        hn_ref[0] = h0n
        hn_ref[1] = h1n
        cn_ref[0] = c0n
        cn_ref[1] = c1n


def _lstm(x, h0, c0, wi0, wh0, b0, wi1, wh1, b1):
    full = lambda shape: pl.BlockSpec(shape, lambda t: (0,) * len(shape))
    return pl.pallas_call(
        _lstm_body,
        grid=(_T,),
        in_specs=[
            pl.BlockSpec((1, _B, _H), lambda t: (t, 0, 0)),
            full((_L, _B, _H)),
            full((_L, _B, _H)),
            full((_H, 4 * _H)),
            full((_H, 4 * _H)),
            full((1, 4 * _H)),
            full((_H, 4 * _H)),
            full((_H, 4 * _H)),
            full((1, 4 * _H)),
        ],
        out_specs=[
            pl.BlockSpec((1, _B, _H), lambda t: (t, 0, 0)),
            full((_L, _B, _H)),
            full((_L, _B, _H)),
        ],
        out_shape=[
            jax.ShapeDtypeStruct((_T, _B, _H), jnp.float32),
            jax.ShapeDtypeStruct((_L, _B, _H), jnp.float32),
            jax.ShapeDtypeStruct((_L, _B, _H), jnp.float32),
        ],
        scratch_shapes=[pltpu.VMEM((_B, _H), jnp.float32)] * 4,
        compiler_params=pltpu.CompilerParams(
            dimension_semantics=("arbitrary",)),
    )(x, h0, c0, wi0, wh0, b0, wi1, wh1, b1)


def kernel(batch_input, h0, c0, table,
           Wih0, Whh0, bih0, bhh0, Wih1, Whh1, bih1, bhh1):
    idx = jnp.transpose(batch_input).astype(jnp.int32)
    idx3 = idx.reshape(_NW, _NCH, _CH)
    emb = _sc_gather(table, idx3)
    x = emb.reshape(_T, _B, _H)
    b0 = (bih0 + bhh0).reshape(1, 4 * _H)
    b1 = (bih1 + bhh1).reshape(1, 4 * _H)
    out_t, hn, cn = _lstm(x, h0, c0,
                          Wih0.T, Whh0.T, b0,
                          Wih1.T, Whh1.T, b1)
    return jnp.transpose(out_t, (1, 0, 2)), hn, cn


# speedup vs baseline: 17.2210x; 1.2239x over previous
import functools

import jax
import jax.numpy as jnp
from jax import lax
from jax.experimental import pallas as pl
from jax.experimental.pallas import tpu as pltpu
from jax.experimental.pallas import tpu_sc as plsc

_V = 1000000
_H = 128
_B = 1024
_T = 200
_L = 2

_NW = 32
_CH = 128
_NCH = (_B * _T) // (_NW * _CH)


def _sc_gather(table, idx3):
    mesh = plsc.VectorSubcoreMesh(core_axis_name="c", subcore_axis_name="s")

    @functools.partial(
        pl.kernel,
        mesh=mesh,
        out_type=jax.ShapeDtypeStruct((_B * _T, _H), jnp.float32),
        scratch_types=[
            pltpu.VMEM((_NCH, _CH), jnp.int32),
            pltpu.VMEM((_CH, _H), jnp.float32),
            pltpu.SemaphoreType.DMA,
        ],
    )
    def gather_k(table_hbm, idx_hbm, out_hbm, idx_v, buf_v, sem):
        wid = lax.axis_index("s") * 2 + lax.axis_index("c")
        pltpu.sync_copy(idx_hbm.at[wid], idx_v)
        base = wid * (_NCH * _CH)

        def body(j, carry):
            pltpu.async_copy(table_hbm.at[idx_v.at[j]], buf_v, sem).wait()
            pltpu.sync_copy(buf_v, out_hbm.at[pl.ds(base + j * _CH, _CH)])
            return carry

        lax.fori_loop(0, _NCH, body, 0)

    return gather_k(table, idx3)


_S = 8
_TB = _T // _S


def _lstm_body(x_ref, h0_ref, c0_ref, wi0_ref, wh0_ref, b0_ref,
               wi1_ref, wh1_ref, b1_ref,
               out_ref, hn_ref, cn_ref,
               h0s, c0s, h1s, c1s):
    tb = pl.program_id(0)

    @pl.when(tb == 0)
    def _init():
        h0s[...] = h0_ref[0]
        c0s[...] = c0_ref[0]
        h1s[...] = h0_ref[1]
        c1s[...] = c0_ref[1]

    def cell(xt, hs, cs, wi, wh, b):
        g = jnp.dot(xt, wi, preferred_element_type=jnp.float32)
        g = g + jnp.dot(hs, wh, preferred_element_type=jnp.float32)
        g = g + b
        i = jax.nn.sigmoid(g[:, 0 * _H:1 * _H])
        f = jax.nn.sigmoid(g[:, 1 * _H:2 * _H])
        gg = jnp.tanh(g[:, 2 * _H:3 * _H])
        o = jax.nn.sigmoid(g[:, 3 * _H:4 * _H])
        c = f * cs + i * gg
        h = o * jnp.tanh(c)
        return h, c

    h0v, c0v = h0s[...], c0s[...]
    h1v, c1v = h1s[...], c1s[...]
    for i in range(_S):
        h0v, c0v = cell(x_ref[i], h0v, c0v, wi0_ref[...], wh0_ref[...],
                        b0_ref[...])
        h1v, c1v = cell(h0v, h1v, c1v, wi1_ref[...], wh1_ref[...],
                        b1_ref[...])
        out_ref[:, i, :] = h1v
    h0s[...] = h0v
    c0s[...] = c0v
    h1s[...] = h1v
    c1s[...] = c1v

    @pl.when(tb == _TB - 1)
    def _fin():
        hn_ref[0] = h0v
        hn_ref[1] = h1v
        cn_ref[0] = c0v
        cn_ref[1] = c1v


def _lstm(x, h0, c0, wi0, wh0, b0, wi1, wh1, b1):
    full = lambda shape: pl.BlockSpec(shape, lambda t: (0,) * len(shape))
    return pl.pallas_call(
        _lstm_body,
        grid=(_TB,),
        in_specs=[
            pl.BlockSpec((_S, _B, _H), lambda t: (t, 0, 0)),
            full((_L, _B, _H)),
            full((_L, _B, _H)),
            full((_H, 4 * _H)),
            full((_H, 4 * _H)),
            full((1, 4 * _H)),
            full((_H, 4 * _H)),
            full((_H, 4 * _H)),
            full((1, 4 * _H)),
        ],
        out_specs=[
            pl.BlockSpec((_B, _S, _H), lambda t: (0, t, 0)),
            full((_L, _B, _H)),
            full((_L, _B, _H)),
        ],
        out_shape=[
            jax.ShapeDtypeStruct((_B, _T, _H), jnp.float32),
            jax.ShapeDtypeStruct((_L, _B, _H), jnp.float32),
            jax.ShapeDtypeStruct((_L, _B, _H), jnp.float32),
        ],
        scratch_shapes=[pltpu.VMEM((_B, _H), jnp.float32)] * 4,
        compiler_params=pltpu.CompilerParams(
            dimension_semantics=("arbitrary",)),
    )(x, h0, c0, wi0, wh0, b0, wi1, wh1, b1)


def kernel(batch_input, h0, c0, table,
           Wih0, Whh0, bih0, bhh0, Wih1, Whh1, bih1, bhh1):
    idx = jnp.transpose(batch_input).astype(jnp.int32)
    idx3 = idx.reshape(_NW, _NCH, _CH)
    emb = _sc_gather(table, idx3)
    x = emb.reshape(_T, _B, _H)
    b0 = (bih0 + bhh0).reshape(1, 4 * _H)
    b1 = (bih1 + bhh1).reshape(1, 4 * _H)
    out, hn, cn = _lstm(x, h0, c0,
                        Wih0.T, Whh0.T, b0,
                        Wih1.T, Whh1.T, b1)
    return out, hn, cn


# speedup vs baseline: 19.4281x; 1.1282x over previous
import functools

import jax
import jax.numpy as jnp
from jax import lax
from jax.experimental import pallas as pl
from jax.experimental.pallas import tpu as pltpu
from jax.experimental.pallas import tpu_sc as plsc

_V = 1000000
_H = 128
_B = 1024
_T = 200
_L = 2

_NW = 32
_CH = 128
_NCH = (_B * _T) // (_NW * _CH)


def _sc_gather(table, idx3):
    mesh = plsc.VectorSubcoreMesh(core_axis_name="c", subcore_axis_name="s")

    @functools.partial(
        pl.kernel,
        mesh=mesh,
        out_type=jax.ShapeDtypeStruct((_B * _T, _H), jnp.float32),
        scratch_types=[
            pltpu.VMEM((_NCH, _CH), jnp.int32),
            pltpu.VMEM((_CH, _H), jnp.float32),
            pltpu.SemaphoreType.DMA,
        ],
    )
    def gather_k(table_hbm, idx_hbm, out_hbm, idx_v, buf_v, sem):
        wid = lax.axis_index("s") * 2 + lax.axis_index("c")
        pltpu.sync_copy(idx_hbm.at[wid], idx_v)
        base = wid * (_NCH * _CH)

        def body(j, carry):
            pltpu.async_copy(table_hbm.at[idx_v.at[j]], buf_v, sem).wait()
            pltpu.sync_copy(buf_v, out_hbm.at[pl.ds(base + j * _CH, _CH)])
            return carry

        lax.fori_loop(0, _NCH, body, 0)

    return gather_k(table, idx3)


_S = 8
_TB = _T // _S


def _lstm_body(x_ref, h0_ref, c0_ref, w0_ref, b0_ref,
               w1_ref, b1_ref,
               out_ref, hn_ref, cn_ref,
               h0s, c0s, h1s, c1s):
    tb = pl.program_id(0)

    @pl.when(tb == 0)
    def _init():
        h0s[...] = h0_ref[0]
        c0s[...] = c0_ref[0]
        h1s[...] = h0_ref[1]
        c1s[...] = c0_ref[1]

    def cell(xt, hs, cs, w, b):
        z = jnp.concatenate([xt, hs], axis=1)
        g = jnp.dot(z, w, preferred_element_type=jnp.float32) + b
        i = jax.nn.sigmoid(g[:, 0 * _H:1 * _H])
        f = jax.nn.sigmoid(g[:, 1 * _H:2 * _H])
        gg = jnp.tanh(g[:, 2 * _H:3 * _H])
        o = jax.nn.sigmoid(g[:, 3 * _H:4 * _H])
        c = f * cs + i * gg
        h = o * jnp.tanh(c)
        return h, c

    h0v, c0v = h0s[...], c0s[...]
    h1v, c1v = h1s[...], c1s[...]
    for i in range(_S):
        h0v, c0v = cell(x_ref[i], h0v, c0v, w0_ref[...], b0_ref[...])
        h1v, c1v = cell(h0v, h1v, c1v, w1_ref[...], b1_ref[...])
        out_ref[:, i, :] = h1v
    h0s[...] = h0v
    c0s[...] = c0v
    h1s[...] = h1v
    c1s[...] = c1v

    @pl.when(tb == _TB - 1)
    def _fin():
        hn_ref[0] = h0v
        hn_ref[1] = h1v
        cn_ref[0] = c0v
        cn_ref[1] = c1v


def _lstm(x, h0, c0, w0, b0, w1, b1):
    full = lambda shape: pl.BlockSpec(shape, lambda t: (0,) * len(shape))
    return pl.pallas_call(
        _lstm_body,
        grid=(_TB,),
        in_specs=[
            pl.BlockSpec((_S, _B, _H), lambda t: (t, 0, 0)),
            full((_L, _B, _H)),
            full((_L, _B, _H)),
            full((2 * _H, 4 * _H)),
            full((1, 4 * _H)),
            full((2 * _H, 4 * _H)),
            full((1, 4 * _H)),
        ],
        out_specs=[
            pl.BlockSpec((_B, _S, _H), lambda t: (0, t, 0)),
            full((_L, _B, _H)),
            full((_L, _B, _H)),
        ],
        out_shape=[
            jax.ShapeDtypeStruct((_B, _T, _H), jnp.float32),
            jax.ShapeDtypeStruct((_L, _B, _H), jnp.float32),
            jax.ShapeDtypeStruct((_L, _B, _H), jnp.float32),
        ],
        scratch_shapes=[pltpu.VMEM((_B, _H), jnp.float32)] * 4,
        compiler_params=pltpu.CompilerParams(
            dimension_semantics=("arbitrary",)),
    )(x, h0, c0, w0, b0, w1, b1)


def kernel(batch_input, h0, c0, table,
           Wih0, Whh0, bih0, bhh0, Wih1, Whh1, bih1, bhh1):
    idx = jnp.transpose(batch_input).astype(jnp.int32)
    idx3 = idx.reshape(_NW, _NCH, _CH)
    emb = _sc_gather(table, idx3)
    x = emb.reshape(_T, _B, _H)
    b0 = (bih0 + bhh0).reshape(1, 4 * _H)
    b1 = (bih1 + bhh1).reshape(1, 4 * _H)
    w0 = jnp.concatenate([Wih0.T, Whh0.T], axis=0)
    w1 = jnp.concatenate([Wih1.T, Whh1.T], axis=0)
    out, hn, cn = _lstm(x, h0, c0, w0, b0, w1, b1)
    return out, hn, cn
